# baseline (speedup 1.0000x reference)
import functools

import jax
import jax.numpy as jnp
from jax.experimental import pallas as pl
from jax.experimental.pallas import tpu as pltpu


def _copy_kernel(T, D, K, in_ref, out_ref, sems):
    rows = T // K
    copies = [
        pltpu.make_async_copy(
            in_ref.at[pl.ds(k * rows, rows)],
            out_ref.at[pl.ds(k * rows, rows)],
            sems.at[k],
        )
        for k in range(K)
    ]
    for c in copies:
        c.start()
    for c in copies:
        c.wait()


def _make_copy(T, D, K=16):
    return pl.pallas_call(
        functools.partial(_copy_kernel, T, D, K),
        in_specs=[pl.BlockSpec(memory_space=pl.ANY)],
        out_specs=pl.BlockSpec(memory_space=pl.ANY),
        out_shape=jax.ShapeDtypeStruct((T, D), jnp.float32),
        scratch_shapes=[pltpu.SemaphoreType.DMA((K,))],
    )


def kernel(x, table):
    T = x.shape[1]
    D = table.shape[1]
    return _make_copy(T, D)(table)


# speedup vs baseline: 34.8490x; 34.8490x over previous
import functools

import jax
import jax.numpy as jnp
from jax import lax
from jax.experimental import pallas as pl
from jax.experimental.pallas import tpu as pltpu


def _prep_kernel(BR, NB, D, off_ref, base_ref, so_ref, co_ref, b_ref):
    off = off_ref[...]
    even = (lax.broadcasted_iota(jnp.int32, (BR, D), 1) % 2) == 0
    so_ref[...] = jnp.where(even, off, pltpu.roll(off, 1, 1))
    co_ref[...] = jnp.where(even, pltpu.roll(off, D - 1, 1), off)
    base = base_ref[...]
    even3 = (lax.broadcasted_iota(jnp.int32, (NB, 1, D), 2) % 2) == 0
    b_ref[...] = jnp.where(even3, pltpu.roll(base, D - 1, 2), -pltpu.roll(base, 1, 2))


def _rot_kernel(BR, NB, NBUF, so_ref, co_ref, a_ref, b_ref, out_ref, buf_ref, sems):
    i = pl.program_id(0)
    slot = lax.rem(i, NBUF)
    row0 = slot * BR

    def _dma(s, j):
        return pltpu.make_async_copy(
            buf_ref.at[pl.ds(s * BR, BR)],
            out_ref.at[pl.ds(j * BR, BR)],
            sems.at[s],
        )

    @pl.when(i >= NBUF)
    def _wait_oldest():
        _dma(slot, i - NBUF).wait()

    buf_ref[pl.ds(row0, BR)] = a_ref[0] * co_ref[...] + b_ref[0] * so_ref[...]
    _dma(slot, i).start()

    @pl.when(i == NB - 1)
    def _drain_all():
        for k in range(NBUF):
            _dma(k, i).wait()


def _make_prep(BR, NB, D):
    return pl.pallas_call(
        functools.partial(_prep_kernel, BR, NB, D),
        out_shape=[
            jax.ShapeDtypeStruct((BR, D), jnp.float32),
            jax.ShapeDtypeStruct((BR, D), jnp.float32),
            jax.ShapeDtypeStruct((NB, 1, D), jnp.float32),
        ],
    )


def _make_rot(T, D, BR, NBUF=4):
    NB = T // BR
    return pl.pallas_call(
        functools.partial(_rot_kernel, BR, NB, NBUF),
        grid=(NB,),
        in_specs=[
            pl.BlockSpec((BR, D), lambda i: (0, 0)),
            pl.BlockSpec((BR, D), lambda i: (0, 0)),
            pl.BlockSpec((1, 1, D), lambda i: (i, 0, 0)),
            pl.BlockSpec((1, 1, D), lambda i: (i, 0, 0)),
        ],
        out_specs=pl.BlockSpec(memory_space=pl.ANY),
        out_shape=jax.ShapeDtypeStruct((T, D), jnp.float32),
        scratch_shapes=[
            pltpu.VMEM((NBUF * BR, D), jnp.float32),
            pltpu.SemaphoreType.DMA((NBUF,)),
        ],
        compiler_params=pltpu.CompilerParams(
            dimension_semantics=("arbitrary",),
        ),
    )


def kernel(x, table):
    T = x.shape[1]
    D = table.shape[1]
    BR = 512
    NB = T // BR
    off_rows = lax.slice(table, (0, 0), (BR, D))
    base_rows = lax.slice(table, (0, 0), (T, D), (BR, 1))
    a_rows = base_rows.reshape(NB, 1, D)
    so, co, b_rows = _make_prep(BR, NB, D)(off_rows, a_rows)
    return _make_rot(T, D, BR)(so, co, a_rows, b_rows)


# speedup vs baseline: 37.6770x; 1.0811x over previous
import functools

import jax
import jax.numpy as jnp
from jax import lax
from jax.experimental import pallas as pl
from jax.experimental.pallas import tpu as pltpu


def _prep_kernel(BR, NB, D, off_ref, base_ref, so_ref, co_ref, b_ref):
    off = off_ref[...]
    even = (lax.broadcasted_iota(jnp.int32, (BR, D), 1) % 2) == 0
    so_ref[...] = jnp.where(even, off, pltpu.roll(off, 1, 1))
    co_ref[...] = jnp.where(even, pltpu.roll(off, D - 1, 1), off)
    base = base_ref[...]
    even3 = (lax.broadcasted_iota(jnp.int32, (NB, 1, D), 2) % 2) == 0
    b_ref[...] = jnp.where(even3, pltpu.roll(base, D - 1, 2), -pltpu.roll(base, 1, 2))


def _rot_kernel(BR, NB2, NBUF, so_ref, co_ref, a0_ref, b0_ref, a1_ref, b1_ref,
                out_ref, buf0_ref, buf1_ref, sems0, sems1):
    i = pl.program_id(0)
    slot = lax.rem(i, NBUF)
    row0 = slot * BR

    def _dma0(s, j):
        return pltpu.make_async_copy(
            buf0_ref.at[pl.ds(s * BR, BR)],
            out_ref.at[pl.ds(j * BR, BR)],
            sems0.at[s],
        )

    def _dma1(s, j):
        return pltpu.make_async_copy(
            buf1_ref.at[pl.ds(s * BR, BR)],
            out_ref.at[pl.ds((NB2 + j) * BR, BR)],
            sems1.at[s],
        )

    @pl.when(i >= NBUF)
    def _wait_oldest():
        _dma0(slot, i - NBUF).wait()
        _dma1(slot, i - NBUF).wait()

    so = so_ref[...]
    co = co_ref[...]
    buf0_ref[pl.ds(row0, BR)] = a0_ref[0] * co + b0_ref[0] * so
    _dma0(slot, i).start()
    buf1_ref[pl.ds(row0, BR)] = a1_ref[0] * co + b1_ref[0] * so
    _dma1(slot, i).start()

    @pl.when(i == NB2 - 1)
    def _drain_all():
        for k in range(NBUF):
            _dma0(k, i).wait()
            _dma1(k, i).wait()


def _make_prep(BR, NB, D):
    return pl.pallas_call(
        functools.partial(_prep_kernel, BR, NB, D),
        out_shape=[
            jax.ShapeDtypeStruct((BR, D), jnp.float32),
            jax.ShapeDtypeStruct((BR, D), jnp.float32),
            jax.ShapeDtypeStruct((NB, 1, D), jnp.float32),
        ],
    )


def _make_rot(T, D, BR, NBUF):
    NB = T // BR
    NB2 = NB // 2
    return pl.pallas_call(
        functools.partial(_rot_kernel, BR, NB2, NBUF),
        grid=(NB2,),
        in_specs=[
            pl.BlockSpec((BR, D), lambda i: (0, 0)),
            pl.BlockSpec((BR, D), lambda i: (0, 0)),
            pl.BlockSpec((1, 1, D), lambda i: (i, 0, 0)),
            pl.BlockSpec((1, 1, D), lambda i: (i, 0, 0)),
            pl.BlockSpec((1, 1, D), lambda i, NB2=NB2: (i + NB2, 0, 0)),
            pl.BlockSpec((1, 1, D), lambda i, NB2=NB2: (i + NB2, 0, 0)),
        ],
        out_specs=pl.BlockSpec(memory_space=pl.ANY),
        out_shape=jax.ShapeDtypeStruct((T, D), jnp.float32),
        scratch_shapes=[
            pltpu.VMEM((NBUF * BR, D), jnp.float32),
            pltpu.VMEM((NBUF * BR, D), jnp.float32),
            pltpu.SemaphoreType.DMA((NBUF,)),
            pltpu.SemaphoreType.DMA((NBUF,)),
        ],
        compiler_params=pltpu.CompilerParams(
            dimension_semantics=("arbitrary",),
        ),
    )


def kernel(x, table):
    T = x.shape[1]
    D = table.shape[1]
    BR = 256
    NBUF = 4
    NB = T // BR
    off_rows = lax.slice(table, (0, 0), (BR, D))
    base_rows = lax.slice(table, (0, 0), (T, D), (BR, 1))
    a_rows = base_rows.reshape(NB, 1, D)
    so, co, b_rows = _make_prep(BR, NB, D)(off_rows, a_rows)
    return _make_rot(T, D, BR, NBUF)(
        so, co,
        a_rows, b_rows,
        a_rows, b_rows,
    )
